# baseline (speedup 1.0000x reference)
import functools

import jax
import jax.numpy as jnp
from jax.experimental import pallas as pl
from jax.experimental.pallas import tpu as pltpu

_LEAKY = 0.01
_EPS = 1e-5
_TILE = 4096
_VMEM = 56 * 1024 * 1024


def _leaky(h):
    return jnp.where(h >= 0.0, h, jnp.float32(_LEAKY) * h)


def _tree_sum(vals):
    while len(vals) > 1:
        pairs = [vals[i] + vals[i + 1] for i in range(0, len(vals) - 1, 2)]
        if len(vals) % 2:
            pairs.append(vals[-1])
        vals = pairs
    return vals[0]


def _stats(a):
    f, t = a.shape
    sp, qp = [None] * 4, [None] * 4
    for idx in range(t // 128):
        c = a[:, idx * 128:(idx + 1) * 128]
        g = idx % 4
        sp[g] = c if sp[g] is None else sp[g] + c
        qp[g] = c * c if qp[g] is None else qp[g] + c * c
    return (_tree_sum([v for v in sp if v is not None]),
            _tree_sum([v for v in qp if v is not None]))


def _tdot(lhs, rhs):
    return jax.lax.dot_general(lhs, rhs,
                               dimension_numbers=(((0,), (0,)), ((), ())),
                               preferred_element_type=jnp.float32)


def _col(vec_ref):
    f = vec_ref.shape[-1]
    row = vec_ref[...].reshape(1, f)
    eye = (jax.lax.broadcasted_iota(jnp.int32, (f, f), 0) ==
           jax.lax.broadcasted_iota(jnp.int32, (f, f), 1)).astype(jnp.float32)
    return jax.lax.dot_general(eye, row,
                               dimension_numbers=(((1,), (1,)), ((), ())),
                               preferred_element_type=jnp.float32)


def _fused_kernel(x_ref, w1, b1r, g1r, be1r, w2, b2r, g2r, be2r,
                  w3, b3r, g3r, be3r, w4r, b4,
                  out_ref,
                  a1t, a2t, a3t, s1, q1,
                  *, nt, tile, rows_valid, rows_padded):
    i = pl.program_id(0)
    padded = rows_valid != rows_padded
    n = jnp.float32(rows_valid)

    def lane_mask(a, j):
        col = j * tile + jax.lax.broadcasted_iota(jnp.int32, a.shape, 1)
        return jnp.where(col < rows_valid, a, 0.0)

    @pl.when(i == 0)
    def _init():
        s1[...] = jnp.zeros_like(s1)
        q1[...] = jnp.zeros_like(q1)

    @pl.when(i < nt)
    def _stage1():
        h = jax.lax.dot_general(w1[...].astype(jnp.bfloat16),
                                x_ref[...].astype(jnp.bfloat16),
                                dimension_numbers=(((0,), (1,)), ((), ())),
                                preferred_element_type=jnp.float32)
        a = _leaky(h + _col(b1r))
        a1t[i] = a.astype(jnp.bfloat16)
        am = lane_mask(a, i) if padded else a
        ds, dq = _stats(am)
        s1[...] += ds
        q1[...] += dq

    @pl.when(i == nt)
    def _rest():
        def fold(s, q, g_row, be_row):
            mean = s / n
            var = jnp.maximum(q / n - mean * mean, 0.0)
            inv = jax.lax.rsqrt(var + _EPS)
            scale = _col(g_row) * inv
            shift = _col(be_row) - mean * scale
            return scale, shift

        def folded_layer(w_ref, b_row, scale, shift):
            wf = (w_ref[...] * scale).astype(jnp.bfloat16)
            bf = _tdot(w_ref[...], shift) + _col(b_row)
            return wf, bf

        def mid_layer(src, dst, wf, bf):
            s = q = None
            for j in range(nt):
                h = _tdot(wf, src[j]) + bf
                a = _leaky(h)
                dst[j] = a.astype(jnp.bfloat16)
                am = lane_mask(a, j) if padded else a
                ds, dq = _stats(am)
                s = ds if s is None else s + ds
                q = dq if q is None else q + dq
            return (jnp.sum(s, axis=1, keepdims=True),
                    jnp.sum(q, axis=1, keepdims=True))

        s1c = jnp.sum(s1[...], axis=1, keepdims=True)
        q1c = jnp.sum(q1[...], axis=1, keepdims=True)
        sc1, sh1 = fold(s1c, q1c, g1r, be1r)
        w2f, b2f = folded_layer(w2, b2r, sc1, sh1)
        s2c, q2c = mid_layer(a1t, a2t, w2f, b2f)

        sc2, sh2 = fold(s2c, q2c, g2r, be2r)
        w3f, b3f = folded_layer(w3, b3r, sc2, sh2)
        s3c, q3c = mid_layer(a2t, a3t, w3f, b3f)

        sc3, sh3 = fold(s3c, q3c, g3r, be3r)
        w4c = w4r[...]
        w4f = (w4c * sc3).astype(jnp.bfloat16)
        b4f = jnp.sum(sh3 * w4c, axis=0, keepdims=True) + b4[...]

        for j in range(nt):
            z = _tdot(w4f, a3t[j]) + b4f
            out_ref[j] = 1.0 / (1.0 + jnp.exp(-z))


def kernel(x, w1, b1, g1, be1, w2, b2, g2, be2, w3, b3, g3, be3, w4, b4):
    batch, k = x.shape
    f1, f2, f3 = w1.shape[1], w2.shape[1], w3.shape[1]
    tile = _TILE if batch >= _TILE else max(128, -(-batch // 128) * 128)
    nt = -(-batch // tile)
    rows_padded = nt * tile
    if rows_padded != batch:
        x = jnp.pad(x, ((0, rows_padded - batch), (0, 0)))

    full = lambda i: (0, 0)
    rspec = lambda f: pl.BlockSpec((1, f), full)
    vspec = lambda f: pl.BlockSpec((f,), lambda i: (0,))
    fused_fn = functools.partial(_fused_kernel, nt=nt, tile=tile,
                                 rows_valid=batch, rows_padded=rows_padded)
    out = pl.pallas_call(
        fused_fn,
        grid=(nt + 1,),
        in_specs=[
            pl.BlockSpec((tile, k), lambda i: (jnp.minimum(i, nt - 1), 0)),
            pl.BlockSpec((k, f1), full), rspec(f1), vspec(f1), vspec(f1),
            pl.BlockSpec((f1, f2), full), rspec(f2), vspec(f2), vspec(f2),
            pl.BlockSpec((f2, f3), full), rspec(f3), vspec(f3), vspec(f3),
            pl.BlockSpec((f3, 1), full), pl.BlockSpec((1, 1), full),
        ],
        out_specs=pl.BlockSpec((nt, 1, tile), lambda i: (0, 0, 0)),
        out_shape=jax.ShapeDtypeStruct((nt, 1, tile), jnp.float32),
        scratch_shapes=[
            pltpu.VMEM((nt, f1, tile), jnp.bfloat16),
            pltpu.VMEM((nt, f2, tile), jnp.bfloat16),
            pltpu.VMEM((nt, f3, tile), jnp.bfloat16),
            pltpu.VMEM((f1, 128), jnp.float32),
            pltpu.VMEM((f1, 128), jnp.float32),
        ],
        compiler_params=pltpu.CompilerParams(
            dimension_semantics=("arbitrary",),
            vmem_limit_bytes=_VMEM),
    )(x, w1, b1, g1, be1, w2, b2, g2, be2, w3, b3, g3, be3, w4, b4)
    return out.reshape(rows_padded, 1)[:batch]


# speedup vs baseline: 1.1628x; 1.1628x over previous
import functools

import jax
import jax.numpy as jnp
from jax.experimental import pallas as pl
from jax.experimental.pallas import tpu as pltpu

_LEAKY = 0.01
_EPS = 1e-5
_CTILE = 4096
_DTILE = 8192
_VMEM = 56 * 1024 * 1024


def _leaky(h):
    return jnp.maximum(h, jnp.float32(_LEAKY) * h)


def _tree_sum(vals):
    while len(vals) > 1:
        pairs = [vals[i] + vals[i + 1] for i in range(0, len(vals) - 1, 2)]
        if len(vals) % 2:
            pairs.append(vals[-1])
        vals = pairs
    return vals[0]


def _stats(a):
    f, t = a.shape
    chunks = [a[:, c:c + 128] for c in range(0, t, 128)]
    return _tree_sum(chunks), _tree_sum([c * c for c in chunks])


def _tdot(lhs, rhs):
    return jax.lax.dot_general(lhs, rhs,
                               dimension_numbers=(((0,), (0,)), ((), ())),
                               preferred_element_type=jnp.float32)


def _col(row_ref):
    f = row_ref.shape[1]
    eye = (jax.lax.broadcasted_iota(jnp.int32, (f, f), 0) ==
           jax.lax.broadcasted_iota(jnp.int32, (f, f), 1)).astype(jnp.float32)
    return jax.lax.dot_general(eye, row_ref[...],
                               dimension_numbers=(((1,), (1,)), ((), ())),
                               preferred_element_type=jnp.float32)


def _fused_kernel(x_ref, w1, b1r, g1r, be1r, w2, b2r, g2r, be2r,
                  w3, b3r, g3r, be3r, w4r, b4,
                  out_ref,
                  a1t, a2t, a3t, s1, q1,
                  *, ndma, sub, nt, ctile, rows_valid, rows_padded):
    i = pl.program_id(0)
    padded = rows_valid != rows_padded
    n = jnp.float32(rows_valid)

    def lane_mask(a, j):
        col = j * ctile + jax.lax.broadcasted_iota(jnp.int32, a.shape, 1)
        return jnp.where(col < rows_valid, a, 0.0)

    @pl.when(i == 0)
    def _init():
        s1[...] = jnp.zeros_like(s1)
        q1[...] = jnp.zeros_like(q1)

    @pl.when(i < ndma)
    def _stage1():
        w1b = w1[...].astype(jnp.bfloat16)
        b1c = _col(b1r)
        for h in range(sub):
            xb = x_ref[h * ctile:(h + 1) * ctile, :].astype(jnp.bfloat16)
            hh = jax.lax.dot_general(
                w1b, xb, dimension_numbers=(((0,), (1,)), ((), ())),
                preferred_element_type=jnp.float32)
            a = _leaky(hh + b1c)
            a1t[i * sub + h] = a.astype(jnp.bfloat16)
            am = lane_mask(a, i * sub + h) if padded else a
            ds, dq = _stats(am)
            s1[...] += ds
            q1[...] += dq

    @pl.when(i == ndma)
    def _rest():
        def fold(s, q, g_row, be_row):
            mean = s / n
            var = jnp.maximum(q / n - mean * mean, 0.0)
            inv = jax.lax.rsqrt(var + _EPS)
            scale = _col(g_row) * inv
            shift = _col(be_row) - mean * scale
            return scale, shift

        def folded_layer(w_ref, b_row, scale, shift):
            wf = (w_ref[...] * scale).astype(jnp.bfloat16)
            bf = _tdot(w_ref[...], shift) + _col(b_row)
            return wf, bf

        def mid_layer(src, dst, wf, bf):
            s = q = None
            for j in range(nt):
                hh = _tdot(wf, src[j]) + bf
                a = _leaky(hh)
                dst[j] = a.astype(jnp.bfloat16)
                am = lane_mask(a, j) if padded else a
                ds, dq = _stats(am)
                s = ds if s is None else s + ds
                q = dq if q is None else q + dq
            return (jnp.sum(s, axis=1, keepdims=True),
                    jnp.sum(q, axis=1, keepdims=True))

        s1c = jnp.sum(s1[...], axis=1, keepdims=True)
        q1c = jnp.sum(q1[...], axis=1, keepdims=True)
        sc1, sh1 = fold(s1c, q1c, g1r, be1r)
        w2f, b2f = folded_layer(w2, b2r, sc1, sh1)
        s2c, q2c = mid_layer(a1t, a2t, w2f, b2f)

        sc2, sh2 = fold(s2c, q2c, g2r, be2r)
        w3f, b3f = folded_layer(w3, b3r, sc2, sh2)
        s3c, q3c = mid_layer(a2t, a3t, w3f, b3f)

        sc3, sh3 = fold(s3c, q3c, g3r, be3r)
        w4c = _col(w4r)
        w4f = (w4c * sc3).astype(jnp.bfloat16)
        b4f = jnp.sum(sh3 * w4c, axis=0, keepdims=True) + b4[...]

        for j in range(nt):
            z = _tdot(w4f, a3t[j]) + b4f
            out_ref[j] = 1.0 / (1.0 + jnp.exp(-z))


def kernel(x, w1, b1, g1, be1, w2, b2, g2, be2, w3, b3, g3, be3, w4, b4):
    batch, k = x.shape
    f1, f2, f3 = w1.shape[1], w2.shape[1], w3.shape[1]
    if batch >= _DTILE:
        dtile = _DTILE
        ctile = _CTILE
    else:
        dtile = ctile = max(128, -(-batch // 128) * 128)
    sub = dtile // ctile
    ndma = -(-batch // dtile)
    rows_padded = ndma * dtile
    nt = rows_padded // ctile
    if rows_padded != batch:
        x = jnp.pad(x, ((0, rows_padded - batch), (0, 0)))

    full = lambda i: (0, 0)
    rspec = lambda f: pl.BlockSpec((1, f), full)
    fused_fn = functools.partial(_fused_kernel, ndma=ndma, sub=sub, nt=nt,
                                 ctile=ctile, rows_valid=batch,
                                 rows_padded=rows_padded)
    out = pl.pallas_call(
        fused_fn,
        grid=(ndma + 1,),
        in_specs=[
            pl.BlockSpec((dtile, k), lambda i: (jnp.minimum(i, ndma - 1), 0)),
            pl.BlockSpec((k, f1), full), rspec(f1), rspec(f1), rspec(f1),
            pl.BlockSpec((f1, f2), full), rspec(f2), rspec(f2), rspec(f2),
            pl.BlockSpec((f2, f3), full), rspec(f3), rspec(f3), rspec(f3),
            rspec(f3), pl.BlockSpec((1, 1), full),
        ],
        out_specs=pl.BlockSpec((nt, 1, ctile), lambda i: (0, 0, 0)),
        out_shape=jax.ShapeDtypeStruct((nt, 1, ctile), jnp.float32),
        scratch_shapes=[
            pltpu.VMEM((nt, f1, ctile), jnp.bfloat16),
            pltpu.VMEM((nt, f2, ctile), jnp.bfloat16),
            pltpu.VMEM((nt, f3, ctile), jnp.bfloat16),
            pltpu.VMEM((f1, 128), jnp.float32),
            pltpu.VMEM((f1, 128), jnp.float32),
        ],
        compiler_params=pltpu.CompilerParams(
            dimension_semantics=("arbitrary",),
            vmem_limit_bytes=_VMEM),
    )(x, w1, b1, g1.reshape(1, f1), be1.reshape(1, f1),
      w2, b2, g2.reshape(1, f2), be2.reshape(1, f2),
      w3, b3, g3.reshape(1, f3), be3.reshape(1, f3),
      w4.reshape(1, f3), b4)
    return out.reshape(rows_padded, 1)[:batch]


# speedup vs baseline: 1.3233x; 1.1380x over previous
import functools

import jax
import jax.numpy as jnp
from jax.experimental import pallas as pl
from jax.experimental.pallas import tpu as pltpu

_LEAKY = 0.01
_EPS = 1e-5
_CTILE = 4096
_DTILE = 8192
_VMEM = 56 * 1024 * 1024


def _leaky(h):
    return jnp.maximum(h, jnp.float32(_LEAKY) * h)


def _tree_sum(vals):
    while len(vals) > 1:
        pairs = [vals[i] + vals[i + 1] for i in range(0, len(vals) - 1, 2)]
        if len(vals) % 2:
            pairs.append(vals[-1])
        vals = pairs
    return vals[0]


def _stats(a):
    f, t = a.shape
    chunks = [a[:, c:c + 128] for c in range(0, t, 128)]
    return _tree_sum(chunks), _tree_sum([c * c for c in chunks])


def _tdot(lhs, rhs):
    return jax.lax.dot_general(lhs, rhs,
                               dimension_numbers=(((0,), (0,)), ((), ())),
                               preferred_element_type=jnp.float32)


def _eye(f):
    return (jax.lax.broadcasted_iota(jnp.int32, (f, f), 0) ==
            jax.lax.broadcasted_iota(jnp.int32, (f, f), 1)).astype(jnp.float32)


def _col(row_ref):
    f = row_ref.shape[1]
    return jax.lax.dot_general(_eye(f), row_ref[...],
                               dimension_numbers=(((1,), (1,)), ((), ())),
                               preferred_element_type=jnp.float32)


def _row(col):
    f = col.shape[0]
    return jax.lax.dot_general(col, _eye(f),
                               dimension_numbers=(((0,), (0,)), ((), ())),
                               preferred_element_type=jnp.float32)


def _fused_kernel(x_ref, w1, b1r, g1r, be1r, w2, b2r, g2r, be2r,
                  w3, b3r, g3r, be3r, w4r, b4,
                  out_ref,
                  a1t, a2t, a3t, s1, q1,
                  *, ndma, sub, nt, ctile, rows_valid, rows_padded):
    i = pl.program_id(0)
    padded = rows_valid != rows_padded
    n = jnp.float32(rows_valid)

    def lane_mask(a, j):
        col = j * ctile + jax.lax.broadcasted_iota(jnp.int32, a.shape, 1)
        return jnp.where(col < rows_valid, a, 0.0)

    @pl.when(i == 0)
    def _init():
        s1[...] = jnp.zeros_like(s1)
        q1[...] = jnp.zeros_like(q1)

    @pl.when(i < ndma)
    def _stage1():
        w1b = w1[...].astype(jnp.bfloat16)
        b1c = _col(b1r)
        for h in range(sub):
            xb = x_ref[h * ctile:(h + 1) * ctile, :].astype(jnp.bfloat16)
            hh = jax.lax.dot_general(
                w1b, xb, dimension_numbers=(((1,), (1,)), ((), ())),
                preferred_element_type=jnp.float32)
            a = _leaky(hh + b1c)
            a1t[i * sub + h] = a.astype(jnp.bfloat16)
            am = lane_mask(a, i * sub + h) if padded else a
            ds, dq = _stats(am)
            s1[...] += ds
            q1[...] += dq

    @pl.when(i == ndma)
    def _rest():
        def fold(s, q, g_row, be_row):
            mean = s / n
            var = jnp.maximum(q / n - mean * mean, 0.0)
            inv = jax.lax.rsqrt(var + _EPS)
            scale = _col(g_row) * inv
            shift = _col(be_row) - mean * scale
            return scale, shift

        def folded_layer(w_ref, b_row, scale, shift):
            wf = (w_ref[...] * _row(scale)).astype(jnp.bfloat16)
            bf = jax.lax.dot_general(
                w_ref[...], shift,
                dimension_numbers=(((1,), (0,)), ((), ())),
                preferred_element_type=jnp.float32) + _col(b_row)
            return wf, bf

        def mid_layer(src, dst, wf, bf):
            s = q = None
            for j in range(nt):
                hh = jax.lax.dot_general(
                    wf, src[j], dimension_numbers=(((1,), (0,)), ((), ())),
                    preferred_element_type=jnp.float32) + bf
                a = _leaky(hh)
                dst[j] = a.astype(jnp.bfloat16)
                am = lane_mask(a, j) if padded else a
                ds, dq = _stats(am)
                s = ds if s is None else s + ds
                q = dq if q is None else q + dq
            return (jnp.sum(s, axis=1, keepdims=True),
                    jnp.sum(q, axis=1, keepdims=True))

        s1c = jnp.sum(s1[...], axis=1, keepdims=True)
        q1c = jnp.sum(q1[...], axis=1, keepdims=True)
        sc1, sh1 = fold(s1c, q1c, g1r, be1r)
        w2f, b2f = folded_layer(w2, b2r, sc1, sh1)
        s2c, q2c = mid_layer(a1t, a2t, w2f, b2f)

        sc2, sh2 = fold(s2c, q2c, g2r, be2r)
        w3f, b3f = folded_layer(w3, b3r, sc2, sh2)
        s3c, q3c = mid_layer(a2t, a3t, w3f, b3f)

        sc3, sh3 = fold(s3c, q3c, g3r, be3r)
        w4c = _col(w4r)
        w4f = (w4c * sc3).astype(jnp.bfloat16)
        b4f = jnp.sum(sh3 * w4c, axis=0, keepdims=True) + b4[...]

        for j in range(nt):
            z = _tdot(w4f, a3t[j]) + b4f
            out_ref[j] = 1.0 / (1.0 + jnp.exp(-z))


def kernel(x, w1, b1, g1, be1, w2, b2, g2, be2, w3, b3, g3, be3, w4, b4):
    batch, k = x.shape
    f1, f2, f3 = w1.shape[1], w2.shape[1], w3.shape[1]
    if batch >= _DTILE:
        dtile = _DTILE
        ctile = _CTILE
    else:
        dtile = ctile = max(128, -(-batch // 128) * 128)
    sub = dtile // ctile
    ndma = -(-batch // dtile)
    rows_padded = ndma * dtile
    nt = rows_padded // ctile
    if rows_padded != batch:
        x = jnp.pad(x, ((0, rows_padded - batch), (0, 0)))

    full = lambda i: (0, 0)
    rspec = lambda f: pl.BlockSpec((1, f), full)
    fused_fn = functools.partial(_fused_kernel, ndma=ndma, sub=sub, nt=nt,
                                 ctile=ctile, rows_valid=batch,
                                 rows_padded=rows_padded)
    out = pl.pallas_call(
        fused_fn,
        grid=(ndma + 1,),
        in_specs=[
            pl.BlockSpec((dtile, k), lambda i: (jnp.minimum(i, ndma - 1), 0)),
            pl.BlockSpec((f1, k), full), rspec(f1), rspec(f1), rspec(f1),
            pl.BlockSpec((f2, f1), full), rspec(f2), rspec(f2), rspec(f2),
            pl.BlockSpec((f3, f2), full), rspec(f3), rspec(f3), rspec(f3),
            rspec(f3), pl.BlockSpec((1, 1), full),
        ],
        out_specs=pl.BlockSpec((nt, 1, ctile), lambda i: (0, 0, 0)),
        out_shape=jax.ShapeDtypeStruct((nt, 1, ctile), jnp.float32),
        scratch_shapes=[
            pltpu.VMEM((nt, f1, ctile), jnp.bfloat16),
            pltpu.VMEM((nt, f2, ctile), jnp.bfloat16),
            pltpu.VMEM((nt, f3, ctile), jnp.bfloat16),
            pltpu.VMEM((f1, 128), jnp.float32),
            pltpu.VMEM((f1, 128), jnp.float32),
        ],
        compiler_params=pltpu.CompilerParams(
            dimension_semantics=("arbitrary",),
            vmem_limit_bytes=_VMEM),
    )(x, w1.T, b1, g1.reshape(1, f1), be1.reshape(1, f1),
      w2.T, b2, g2.reshape(1, f2), be2.reshape(1, f2),
      w3.T, b3, g3.reshape(1, f3), be3.reshape(1, f3),
      w4.reshape(1, f3), b4)
    return out.reshape(rows_padded, 1)[:batch]
